# baseline (speedup 1.0000x reference)
import jax
import jax.numpy as jnp
from jax.experimental import pallas as pl
from jax.experimental.pallas import tpu as pltpu

_B, _N, _L = 4, 4096, 1024
_BLK = 2048
_CH = 256


def _scan_body(x_ref, o_ref, carry_ref):
    i = pl.program_id(1)

    @pl.when(i == 0)
    def _():
        carry_ref[...] = jnp.zeros_like(carry_ref)

    rows = jax.lax.broadcasted_iota(jnp.int32, (_CH, _CH), 0)
    cols = jax.lax.broadcasted_iota(jnp.int32, (_CH, _CH), 1)
    tri = (cols < rows).astype(jnp.float32)

    tot = carry_ref[...]
    for c in range(_BLK // _CH):
        xc = x_ref[0, pl.ds(c * _CH, _CH), :]
        excl = jnp.dot(tri, xc, preferred_element_type=jnp.float32)
        o_ref[0, pl.ds(c * _CH, _CH), :] = excl + tot
        tot = tot + jnp.sum(xc, axis=0, keepdims=True)
    carry_ref[...] = tot


def kernel(x):
    return pl.pallas_call(
        _scan_body,
        grid=(_B, _N // _BLK),
        in_specs=[pl.BlockSpec((1, _BLK, _L), lambda b, i: (b, i, 0))],
        out_specs=pl.BlockSpec((1, _BLK, _L), lambda b, i: (b, i, 0)),
        out_shape=jax.ShapeDtypeStruct((_B, _N, _L), jnp.float32),
        scratch_shapes=[pltpu.VMEM((1, _L), jnp.float32)],
        compiler_params=pltpu.CompilerParams(
            dimension_semantics=("parallel", "arbitrary")
        ),
    )(x)


# speedup vs baseline: 1.0083x; 1.0083x over previous
import jax
import jax.numpy as jnp
from jax.experimental import pallas as pl
from jax.experimental.pallas import tpu as pltpu

_B, _N, _L = 4, 4096, 1024
_R = _B * _N
_CH = 256
_MAXT = 2048
_SCHED = [512, 512, 1024, 2048, 2048, 2048, 2048, 2048, 2048, 1024, 512, 512]
assert sum(_SCHED) == _R


def _scan_tile(in_buf, out_buf, slot, rows, carry):
    rows_i = jax.lax.broadcasted_iota(jnp.int32, (_CH, _CH), 0)
    cols_i = jax.lax.broadcasted_iota(jnp.int32, (_CH, _CH), 1)
    tri = (cols_i < rows_i).astype(jnp.float32)
    for c in range(rows // _CH):
        xc = in_buf[slot, pl.ds(c * _CH, _CH), :]
        excl = jnp.dot(tri, xc, preferred_element_type=jnp.float32)
        out_buf[slot, pl.ds(c * _CH, _CH), :] = excl + carry
        carry = carry + jnp.sum(xc, axis=0, keepdims=True)
    return carry


def _body(x_ref, o_ref, in_buf, out_buf, rsem, wsem):
    T = len(_SCHED)
    starts = [0]
    for r in _SCHED:
        starts.append(starts[-1] + r)

    def rd(t):
        return pltpu.make_async_copy(
            x_ref.at[pl.ds(starts[t], _SCHED[t]), :],
            in_buf.at[t % 2, pl.ds(0, _SCHED[t]), :],
            rsem.at[t % 2],
        )

    def wr(t):
        return pltpu.make_async_copy(
            out_buf.at[t % 2, pl.ds(0, _SCHED[t]), :],
            o_ref.at[pl.ds(starts[t], _SCHED[t]), :],
            wsem.at[t % 2],
        )

    rd(0).start()
    rd(1).start()
    carry = jnp.zeros((1, _L), jnp.float32)
    for t in range(T):
        s = t % 2
        rd(t).wait()
        if t >= 2:
            wr(t - 2).wait()
        if starts[t] % _N == 0:
            carry = jnp.zeros((1, _L), jnp.float32)
        carry = _scan_tile(in_buf, out_buf, s, _SCHED[t], carry)
        wr(t).start()
        if t + 2 < T:
            rd(t + 2).start()
    wr(T - 2).wait()
    wr(T - 1).wait()


def kernel(x):
    x2 = x.reshape(_R, _L)
    out = pl.pallas_call(
        _body,
        in_specs=[pl.BlockSpec(memory_space=pl.ANY)],
        out_specs=pl.BlockSpec(memory_space=pl.ANY),
        out_shape=jax.ShapeDtypeStruct((_R, _L), jnp.float32),
        scratch_shapes=[
            pltpu.VMEM((2, _MAXT, _L), jnp.float32),
            pltpu.VMEM((2, _MAXT, _L), jnp.float32),
            pltpu.SemaphoreType.DMA((2,)),
            pltpu.SemaphoreType.DMA((2,)),
        ],
    )(x2)
    return out.reshape(_B, _N, _L)
